# baseline (speedup 1.0000x reference)
import functools

import jax
import jax.numpy as jnp
from jax import lax
from jax.experimental import pallas as pl
from jax.experimental.pallas import tpu as pltpu
from jax.experimental.pallas import tpu_sc as plsc

_INFO = plsc.get_sparse_core_info()
_NC = _INFO.num_cores
_NS = _INFO.num_subcores
_NW = _NC * _NS


def _build(B, V, D):
    b_per_w = B // _NW
    mesh = plsc.VectorSubcoreMesh(core_axis_name="c", subcore_axis_name="s")

    @functools.partial(
        pl.kernel,
        mesh=mesh,
        out_type=jax.ShapeDtypeStruct((B, D), jnp.float32),
        compiler_params=pltpu.CompilerParams(use_tc_tiling_on_sc=False),
        scratch_types=[
            pltpu.VMEM((b_per_w,), jnp.int32),
            pltpu.VMEM((b_per_w, D), jnp.float32),
            pltpu.SemaphoreType.DMA,
        ],
    )
    def k(labels_hbm, table_hbm, out_hbm, idx_v, rows_v, sem):
        wid = lax.axis_index("s") * _NC + lax.axis_index("c")
        base = wid * b_per_w
        pltpu.sync_copy(labels_hbm.at[pl.ds(base, b_per_w)], idx_v)
        pltpu.async_copy(table_hbm.at[idx_v], rows_v, sem).wait()
        pltpu.sync_copy(rows_v, out_hbm.at[pl.ds(base, b_per_w)])

    return k


def kernel(labels, embedding_table):
    B = labels.shape[0]
    V, D = embedding_table.shape
    return _build(B, V, D)(labels.astype(jnp.int32), embedding_table)


# speedup vs baseline: 1.4913x; 1.4913x over previous
import functools

import jax
import jax.numpy as jnp
from jax import lax
from jax.experimental import pallas as pl
from jax.experimental.pallas import tpu as pltpu
from jax.experimental.pallas import tpu_sc as plsc

_INFO = plsc.get_sparse_core_info()
_NC = _INFO.num_cores
_NS = _INFO.num_subcores
_NW = _NC * _NS
_L = 16


def _build(B, V, D):
    b_per_w = B // _NW
    n_blk = b_per_w // _L
    mesh = plsc.VectorSubcoreMesh(core_axis_name="c", subcore_axis_name="s")

    @functools.partial(
        pl.kernel,
        mesh=mesh,
        out_type=jax.ShapeDtypeStruct((B, D), jnp.float32),
        compiler_params=pltpu.CompilerParams(use_tc_tiling_on_sc=True),
        scratch_types=[
            pltpu.VMEM((b_per_w,), jnp.int32),
            pltpu.VMEM((b_per_w, D), jnp.float32),
            pltpu.SemaphoreType.DMA,
        ],
    )
    def k(labels_hbm, table_hbm, out_hbm, idx_v, rows_v, sem):
        wid = lax.axis_index("s") * _NC + lax.axis_index("c")
        base = wid * b_per_w
        pltpu.sync_copy(labels_hbm.at[pl.ds(base, b_per_w)], idx_v)

        def fire(blk, carry):
            v = idx_v[pl.ds(blk * _L, _L)]
            for lane in range(_L):
                r = v[lane]
                pltpu.make_async_copy(
                    table_hbm.at[r], rows_v.at[blk * _L + lane], sem
                ).start()
            return carry

        lax.fori_loop(0, n_blk, fire, 0, unroll=False)
        pltpu.make_async_copy(table_hbm.at[pl.ds(0, b_per_w)], rows_v, sem).wait()
        pltpu.sync_copy(rows_v, out_hbm.at[pl.ds(base, b_per_w)])

    return k


def kernel(labels, embedding_table):
    B = labels.shape[0]
    V, D = embedding_table.shape
    return _build(B, V, D)(labels.astype(jnp.int32), embedding_table)


# speedup vs baseline: 1.8309x; 1.2277x over previous
import functools

import jax
import jax.numpy as jnp
from jax import lax
from jax.experimental import pallas as pl
from jax.experimental.pallas import tpu as pltpu
from jax.experimental.pallas import tpu_sc as plsc

_INFO = plsc.get_sparse_core_info()
_NC = _INFO.num_cores
_NS = _INFO.num_subcores
_NW = _NC * _NS
_L = 16
_CHUNK = 2048


def _build(B, V, D):
    feats_per_w = D // _NW
    n_chunks = B // _CHUNK
    mesh = plsc.VectorSubcoreMesh(core_axis_name="c", subcore_axis_name="s")

    @functools.partial(
        pl.kernel,
        mesh=mesh,
        out_type=jax.ShapeDtypeStruct((D, B), jnp.float32),
        compiler_params=pltpu.CompilerParams(
            use_tc_tiling_on_sc=True, needs_layout_passes=False
        ),
        scratch_types=[
            pltpu.VMEM((V,), jnp.float32),
            pltpu.VMEM((B,), jnp.int32),
            pltpu.VMEM((2, _CHUNK), jnp.float32),
            pltpu.SemaphoreType.DMA,
            pltpu.SemaphoreType.DMA,
        ],
    )
    def k(table_t_hbm, labels_hbm, out_t_hbm, fv, idx_v, st, sem_f, sem_o):
        wid = lax.axis_index("s") * _NC + lax.axis_index("c")
        pltpu.sync_copy(labels_hbm, idx_v)

        def store_copy(f, ch, buf):
            c = wid * feats_per_w + f
            return pltpu.make_async_copy(
                st.at[buf], out_t_hbm.at[c, pl.ds(ch * _CHUNK, _CHUNK)], sem_o
            )

        for f in range(feats_per_w):
            c = wid * feats_per_w + f
            pltpu.make_async_copy(table_t_hbm.at[c], fv, sem_f).start()
            pltpu.make_async_copy(table_t_hbm.at[c], fv, sem_f).wait()
            for ch in range(n_chunks):
                g = f * n_chunks + ch
                buf = g & 1
                if g >= 2:
                    prev = g - 2
                    store_copy(prev // n_chunks, prev % n_chunks, buf).wait()

                def gather(j, carry):
                    ii = idx_v[pl.ds(ch * _CHUNK + j * _L, _L)]
                    st[buf, pl.ds(j * _L, _L)] = plsc.load_gather(fv, [ii])
                    return carry

                lax.fori_loop(0, _CHUNK // _L, gather, 0, unroll=4)
                store_copy(f, ch, buf).start()

        total = feats_per_w * n_chunks
        for g in (total - 2, total - 1):
            store_copy(g // n_chunks, g % n_chunks, g & 1).wait()

    return k


def kernel(labels, embedding_table):
    B = labels.shape[0]
    V, D = embedding_table.shape
    out_t = _build(B, V, D)(embedding_table.T, labels.astype(jnp.int32))
    return out_t.T


# speedup vs baseline: 2.5278x; 1.3806x over previous
import functools

import jax
import jax.numpy as jnp
from jax import lax
from jax.experimental import pallas as pl
from jax.experimental.pallas import tpu as pltpu
from jax.experimental.pallas import tpu_sc as plsc

_INFO = plsc.get_sparse_core_info()
_NC = _INFO.num_cores
_NS = _INFO.num_subcores
_NW = _NC * _NS
_L = 16
_CHUNK = 2048


def _build(B, V, D):
    feats_per_w = D // _NW
    n_chunks = B // _CHUNK
    mesh = plsc.VectorSubcoreMesh(core_axis_name="c", subcore_axis_name="s")

    @functools.partial(
        pl.kernel,
        mesh=mesh,
        out_type=jax.ShapeDtypeStruct((D, B), jnp.float32),
        compiler_params=pltpu.CompilerParams(
            use_tc_tiling_on_sc=True, needs_layout_passes=False
        ),
        scratch_types=[
            pltpu.VMEM((V,), jnp.float32),
            pltpu.VMEM((B,), jnp.int32),
            pltpu.VMEM((2, _CHUNK), jnp.float32),
            pltpu.SemaphoreType.DMA,
            pltpu.SemaphoreType.DMA,
        ],
    )
    def k(table_t_hbm, labels_hbm, out_t_hbm, fv, idx_v, st, sem_f, sem_o):
        wid = lax.axis_index("s") * _NC + lax.axis_index("c")
        pltpu.sync_copy(labels_hbm, idx_v)

        def store_copy(f, ch, buf):
            c = wid * feats_per_w + f
            return pltpu.make_async_copy(
                st.at[buf], out_t_hbm.at[c, pl.ds(ch * _CHUNK, _CHUNK)], sem_o
            )

        for f in range(feats_per_w):
            c = wid * feats_per_w + f
            pltpu.make_async_copy(table_t_hbm.at[c], fv, sem_f).start()
            pltpu.make_async_copy(table_t_hbm.at[c], fv, sem_f).wait()
            for ch in range(n_chunks):
                g = f * n_chunks + ch
                buf = g & 1
                if g >= 2:
                    prev = g - 2
                    store_copy(prev // n_chunks, prev % n_chunks, buf).wait()

                @plsc.parallel_loop(0, _CHUNK // _L, unroll=4)
                def gather(j):
                    ii = idx_v[pl.ds(ch * _CHUNK + j * _L, _L)]
                    st[buf, pl.ds(j * _L, _L)] = plsc.load_gather(fv, [ii])
                store_copy(f, ch, buf).start()

        total = feats_per_w * n_chunks
        for g in (total - 2, total - 1):
            store_copy(g // n_chunks, g % n_chunks, g & 1).wait()

    return k


def kernel(labels, embedding_table):
    B = labels.shape[0]
    V, D = embedding_table.shape
    out_t = _build(B, V, D)(embedding_table.T, labels.astype(jnp.int32))
    return out_t.T


# speedup vs baseline: 2.6470x; 1.0471x over previous
import functools

import jax
import jax.numpy as jnp
from jax import lax
from jax.experimental import pallas as pl
from jax.experimental.pallas import tpu as pltpu
from jax.experimental.pallas import tpu_sc as plsc

_INFO = plsc.get_sparse_core_info()
_NC = _INFO.num_cores
_NS = _INFO.num_subcores
_NW = _NC * _NS
_L = 16
_CHUNK = 4096


def _build(B, V, D):
    feats_per_w = D // _NW
    n_chunks = B // _CHUNK
    mesh = plsc.VectorSubcoreMesh(core_axis_name="c", subcore_axis_name="s")

    @functools.partial(
        pl.kernel,
        mesh=mesh,
        out_type=jax.ShapeDtypeStruct((D, B), jnp.float32),
        compiler_params=pltpu.CompilerParams(
            use_tc_tiling_on_sc=True, needs_layout_passes=False
        ),
        scratch_types=[
            pltpu.VMEM((V,), jnp.float32),
            pltpu.VMEM((B,), jnp.int32),
            pltpu.VMEM((2, _CHUNK), jnp.float32),
            pltpu.SemaphoreType.DMA,
            pltpu.SemaphoreType.DMA,
        ],
    )
    def k(table_t_hbm, labels_hbm, out_t_hbm, fv, idx_v, st, sem_f, sem_o):
        wid = lax.axis_index("s") * _NC + lax.axis_index("c")

        def fv_copy(f):
            c = wid * feats_per_w + f
            return pltpu.make_async_copy(table_t_hbm.at[c], fv, sem_f)

        def store_copy(f, ch, buf):
            c = wid * feats_per_w + f
            return pltpu.make_async_copy(
                st.at[buf], out_t_hbm.at[c, pl.ds(ch * _CHUNK, _CHUNK)], sem_o
            )

        fv_copy(0).start()
        pltpu.sync_copy(labels_hbm, idx_v)
        fv_copy(0).wait()

        for f in range(feats_per_w):
            if f > 0:
                fv_copy(f).start()
                fv_copy(f).wait()
            for ch in range(n_chunks):
                g = f * n_chunks + ch
                buf = g & 1
                if g >= 2:
                    prev = g - 2
                    store_copy(prev // n_chunks, prev % n_chunks, buf).wait()

                @plsc.parallel_loop(0, _CHUNK // _L, unroll=8)
                def gather(j):
                    ii = idx_v[pl.ds(ch * _CHUNK + j * _L, _L)]
                    st[buf, pl.ds(j * _L, _L)] = plsc.load_gather(fv, [ii])

                store_copy(f, ch, buf).start()

        total = feats_per_w * n_chunks
        for g in (total - 2, total - 1):
            store_copy(g // n_chunks, g % n_chunks, g & 1).wait()

    return k


def kernel(labels, embedding_table):
    B = labels.shape[0]
    V, D = embedding_table.shape
    out_t = _build(B, V, D)(embedding_table.T, labels.astype(jnp.int32))
    return out_t.T


# speedup vs baseline: 2.6493x; 1.0009x over previous
import functools

import jax
import jax.numpy as jnp
from jax import lax
from jax.experimental import pallas as pl
from jax.experimental.pallas import tpu as pltpu
from jax.experimental.pallas import tpu_sc as plsc

_INFO = plsc.get_sparse_core_info()
_NC = _INFO.num_cores
_NS = _INFO.num_subcores
_NW = _NC * _NS
_L = 16
_CHUNK = 4096


def _build(B, V, D):
    feats_per_w = D // _NW
    n_chunks = B // _CHUNK
    mesh = plsc.VectorSubcoreMesh(core_axis_name="c", subcore_axis_name="s")

    @functools.partial(
        pl.kernel,
        mesh=mesh,
        out_type=jax.ShapeDtypeStruct((D, B), jnp.float32),
        compiler_params=pltpu.CompilerParams(
            use_tc_tiling_on_sc=True,
            needs_layout_passes=False,
            disable_bounds_checks=True,
            disable_semaphore_checks=True,
        ),
        scratch_types=[
            pltpu.VMEM((V,), jnp.float32),
            pltpu.VMEM((B,), jnp.int32),
            pltpu.VMEM((2, _CHUNK), jnp.float32),
            pltpu.SemaphoreType.DMA,
            pltpu.SemaphoreType.DMA,
        ],
    )
    def k(table_t_hbm, labels_hbm, out_t_hbm, fv, idx_v, st, sem_f, sem_o):
        wid = lax.axis_index("s") * _NC + lax.axis_index("c")

        def fv_copy(f):
            c = wid * feats_per_w + f
            return pltpu.make_async_copy(table_t_hbm.at[c], fv, sem_f)

        def store_copy(f, ch, buf):
            c = wid * feats_per_w + f
            return pltpu.make_async_copy(
                st.at[buf], out_t_hbm.at[c, pl.ds(ch * _CHUNK, _CHUNK)], sem_o
            )

        fv_copy(0).start()
        pltpu.sync_copy(labels_hbm, idx_v)
        fv_copy(0).wait()

        for f in range(feats_per_w):
            if f > 0:
                fv_copy(f).start()
                fv_copy(f).wait()
            for ch in range(n_chunks):
                g = f * n_chunks + ch
                buf = g & 1
                if g >= 2:
                    prev = g - 2
                    store_copy(prev // n_chunks, prev % n_chunks, buf).wait()

                @plsc.parallel_loop(0, _CHUNK // _L, unroll=8)
                def gather(j):
                    ii = idx_v[pl.ds(ch * _CHUNK + j * _L, _L)]
                    st[buf, pl.ds(j * _L, _L)] = plsc.load_gather(fv, [ii])

                store_copy(f, ch, buf).start()

        total = feats_per_w * n_chunks
        for g in (total - 2, total - 1):
            store_copy(g // n_chunks, g % n_chunks, g & 1).wait()

    return k


def kernel(labels, embedding_table):
    B = labels.shape[0]
    V, D = embedding_table.shape
    out_t = _build(B, V, D)(embedding_table.T, labels.astype(jnp.int32))
    return out_t.T
